# baseline (speedup 1.0000x reference)
import functools

import jax
import jax.numpy as jnp
from jax.experimental import pallas as pl
from jax.experimental.pallas import tpu as pltpu

_F32 = jnp.float32
_BF16 = jnp.bfloat16


def _shift_r(v):
    return jnp.concatenate([jnp.zeros_like(v[:, :1]), v[:, :-1]], axis=1)


def _shift_l(v):
    return jnp.concatenate([v[:, 1:], jnp.zeros_like(v[:, :1])], axis=1)


def _body(cw0_ref, cb0_ref, cw1_ref, cb1_ref,
          x_ref, eps_ref, adj_ref,
          w1_ref, b1_ref, w2_ref, b2_ref, w31_ref, b31_ref,
          w21_ref, b21_ref, w22_ref, b22_ref,
          w3_ref, b3_ref, w32_ref, b32_ref, w4_ref, b4_ref,
          g1_ref, g3_ref, g4_ref, g5_ref, fcw_ref, fcb_ref,
          mu_ref, lv_ref, rec_ref, out_ref,
          adjb_s, t1_s, *, C, L, TM, steps):
    def mm(a, b):
        return jnp.dot(a, b, preferred_element_type=_F32)

    def mmw(a, w_ref):
        return mm(a, w_ref[...].astype(_BF16))

    i = pl.program_id(0)

    @pl.when(i < steps)
    def _ae():
        row = i * TM
        adjb_s[pl.ds(row, TM), :] = adj_ref[...].astype(_BF16)

        xr = x_ref[...]
        pro = jnp.full((TM, L), cb0_ref[0], _F32)
        for c in range(C):
            xc = xr[:, c, :]
            pro += (cw0_ref[c, 0] * _shift_r(xc)
                    + cw0_ref[c, 1] * xc
                    + cw0_ref[c, 2] * _shift_l(xc))
        pro = pro.astype(_BF16)
        t1_s[pl.ds(row, TM), :] = mmw(pro, g1_ref).astype(_BF16)

        h = jnp.maximum(mmw(pro, w1_ref) + b1_ref[...], 0.0).astype(_BF16)
        h = jnp.maximum(mmw(h, w2_ref) + b2_ref[...], 0.0).astype(_BF16)
        h = jnp.maximum(mmw(h, w31_ref) + b31_ref[...], 0.0).astype(_BF16)
        mu = mmw(h, w21_ref) + b21_ref[...]
        lv = mmw(h, w22_ref) + b22_ref[...]
        mu_ref[...] = mu
        lv_ref[...] = lv

        z = (eps_ref[...] * jnp.exp(0.5 * lv) + mu).astype(_BF16)
        d = jnp.maximum(mmw(z, w3_ref) + b3_ref[...], 0.0).astype(_BF16)
        d = jnp.maximum(mmw(d, w32_ref) + b32_ref[...], 0.0).astype(_BF16)
        y = mmw(d, w4_ref) + b4_ref[...]
        recon = 0.5 * (jnp.tanh(0.5 * y) + 1.0)

        r_m1 = _shift_r(recon)
        r_p1 = _shift_l(recon)
        for c in range(C):
            rec_ref[:, c, :] = (cw1_ref[c, 0] * r_m1
                                + cw1_ref[c, 1] * recon
                                + cw1_ref[c, 2] * r_p1 + cb1_ref[c])

    @pl.when(i == steps)
    def _gnn():
        h1 = jnp.maximum(mm(adjb_s[...], t1_s[...]), 0.0)
        t2 = mmw(h1.astype(_BF16), g3_ref).astype(_BF16)
        h2 = jnp.maximum(mm(adjb_s[...], t2), 0.0)
        t3 = mmw(h2.astype(_BF16), g4_ref).astype(_BF16)
        h3 = mm(adjb_s[...], t3)
        t4 = mmw(h3.astype(_BF16), g5_ref).astype(_BF16)
        h4 = mm(adjb_s[...], t4)
        logits = mmw(h4.astype(_BF16), fcw_ref) + fcb_ref[...]
        logits = logits - jnp.max(logits, axis=-1, keepdims=True)
        e = jnp.exp(logits)
        out_ref[...] = e * pl.reciprocal(jnp.sum(e, axis=-1, keepdims=True),
                                         approx=True)


def kernel(conv0_w, conv0_b, conv1_w, conv1_b,
           fc1_w, fc1_b, fc2_w, fc2_b, fc31_w, fc31_b,
           fc21_w, fc21_b, fc22_w, fc22_b, fc3_w, fc3_b,
           fc32_w, fc32_b, fc4_w, fc4_b,
           gnn1_w, gnn3_w, gnn4_w, gnn5_w, fc_w, fc_b,
           x, adj, eps):
    N, C, L = x.shape
    n_lat = fc21_w.shape[1]
    n_clusters = fc_w.shape[1]
    Zg = gnn1_w.shape[1]
    H = fc2_w.shape[1]

    TM = 256
    steps = N // TM
    vmem = pltpu.MemorySpace.VMEM
    smem = pltpu.MemorySpace.SMEM

    def full(a):
        return pl.BlockSpec(memory_space=vmem)

    def rows(block, rank3=False):
        if rank3:
            return pl.BlockSpec(block, lambda i: (jnp.minimum(i, steps - 1), 0, 0))
        return pl.BlockSpec(block, lambda i: (jnp.minimum(i, steps - 1), 0))

    weights = (fc1_w, fc1_b, fc2_w, fc2_b, fc31_w, fc31_b,
               fc21_w, fc21_b, fc22_w, fc22_b,
               fc3_w, fc3_b, fc32_w, fc32_b, fc4_w, fc4_b,
               gnn1_w, gnn3_w, gnn4_w, gnn5_w, fc_w, fc_b)
    flops = 2 * N * (12 * L + L * H + 3 * H * H + 2 * H * n_lat + n_lat * H
                     + H * L + 12 * L + L * Zg) \
        + 2 * (4 * N * N * Zg + 3 * N * Zg * Zg + N * Zg * n_clusters)
    bytes_accessed = 4 * (N * C * L + N * n_lat + N * N) \
        + 4 * sum(int(a.size) for a in weights) \
        + 4 * (2 * N * n_lat + N * C * L + N * n_clusters)

    mu, lv, rec, predict = pl.pallas_call(
        functools.partial(_body, C=C, L=L, TM=TM, steps=steps),
        grid=(steps + 1,),
        in_specs=([pl.BlockSpec(memory_space=smem)] * 4
                  + [rows((TM, C, L), rank3=True), rows((TM, n_lat)),
                     rows((TM, N))]
                  + [full(a) for a in weights]),
        out_specs=(rows((TM, n_lat)), rows((TM, n_lat)),
                   rows((TM, C, L), rank3=True),
                   pl.BlockSpec((N, n_clusters), lambda i: (0, 0))),
        out_shape=(jax.ShapeDtypeStruct((N, n_lat), _F32),
                   jax.ShapeDtypeStruct((N, n_lat), _F32),
                   jax.ShapeDtypeStruct((N, C, L), _F32),
                   jax.ShapeDtypeStruct((N, n_clusters), _F32)),
        scratch_shapes=[pltpu.VMEM((N, N), _BF16), pltpu.VMEM((N, Zg), _BF16)],
        compiler_params=pltpu.CompilerParams(
            dimension_semantics=("arbitrary",)),
        cost_estimate=pl.CostEstimate(flops=flops,
                                      transcendentals=N * (n_lat + L + n_clusters),
                                      bytes_accessed=bytes_accessed),
    )(conv0_w, conv0_b, conv1_w, conv1_b, x, eps, adj, *weights)

    return rec, predict, mu, lv
